# Initial kernel scaffold; baseline (speedup 1.0000x reference)
import jax
import jax.numpy as jnp
from jax.experimental import pallas as pl


def kernel(attention, h, edge_index):
    raise NotImplementedError("write your pallas kernel here")


# speedup vs baseline: 25.7975x; 25.7975x over previous
import functools

import jax
import jax.numpy as jnp
from jax import lax
from jax.experimental import pallas as pl
from jax.experimental.pallas import tpu as pltpu
from jax.experimental.pallas import tpu_sc as plsc

_NC = 2
_NS = 16
_L = 16

_BLK = 1024
_GRP = 128
_GPB = _BLK // _GRP


def _bcast_lane(v, lane):
    idx = jnp.full((_L, 1), lane, dtype=jnp.int32)
    dn = lax.GatherDimensionNumbers(
        offset_dims=(), collapsed_slice_dims=(0,), start_index_map=(0,))
    return lax.gather(v, idx, dn, (1,),
                      mode=lax.GatherScatterMode.PROMISE_IN_BOUNDS)


def _body(n_nodes, n_blocks, hr, coli, rowi, attr, out,
          idx2, rowb, attb, rows, acc, sem):
    c = lax.axis_index("c")
    s = lax.axis_index("s")
    rows_per_tile = n_nodes // _NS
    zchunk = 125

    zero = jnp.zeros((_L,), jnp.float32)

    def zloop(i, _):
        for k in range(4):
            rows[i, pl.ds(k * _L, _L)] = zero
        return 0
    lax.fori_loop(0, _GRP, zloop, 0)

    r0 = s * rows_per_tile
    for t in range(rows_per_tile // zchunk):
        pltpu.sync_copy(rows.at[pl.ds(0, zchunk), :],
                        acc.at[pl.ds(r0 + t * zchunk, zchunk), :])
    plsc.subcore_barrier()

    cN = c * n_nodes

    def blk_loop(blk, _):
        base128 = s * (n_blocks * _GPB) + blk * _GPB
        qblock = base128 * (_GRP // 4)
        pltpu.sync_copy(coli.at[pl.ds(base128, _GPB), :], idx2)
        pltpu.sync_copy(rowi.at[pl.ds(base128, _GPB), :], rowb)
        pltpu.sync_copy(attr.at[c, pl.ds(qblock, _BLK // 4), :], attb)

        cNv = jnp.full((_L,), cN, jnp.int32)

        def off_loop(i, _):
            for k in range(8):
                sl = pl.ds(k * _L, _L)
                idx2[i, sl] = idx2[i, sl] + cNv
            return 0
        lax.fori_loop(0, _GPB, off_loop, 0)

        def grp_loop(j, _):
            pltpu.async_copy(hr.at[idx2.at[j]], rows, sem).wait()

            def quad_loop(q, _2):
                av = attb[j * (_GRP // 4) + q, :]
                for jj in range(4):
                    e = q * 4 + jj
                    for hd in range(4):
                        bc = _bcast_lane(av, 4 * jj + hd)
                        sl = pl.ds(hd * _L, _L)
                        rows[e, sl] = rows[e, sl] * bc
                return 0
            lax.fori_loop(0, _GRP // 4, quad_loop, 0)
            pltpu.sync_copy(rows, acc.at[rowb.at[j]], add=True)
            return 0
        lax.fori_loop(0, _GPB, grp_loop, 0)
        return 0
    lax.fori_loop(0, n_blocks, blk_loop, 0)

    plsc.subcore_barrier()
    for t in range(rows_per_tile // zchunk):
        rr = r0 + t * zchunk
        pltpu.sync_copy(acc.at[pl.ds(rr, zchunk), :],
                        out.at[pl.ds(rr, zchunk), pl.ds(c * 64, 64)])


def kernel(attention, h, edge_index):
    E, H = attention.shape
    N, _, D = h.shape
    HD = H * D
    ept = _NS * _BLK
    epad = ((E + ept - 1) // ept) * ept
    n_blocks = epad // ept
    pad = epad - E

    row = edge_index[0].astype(jnp.int32)
    col = edge_index[1].astype(jnp.int32)
    rowp = jnp.pad(row, (0, pad))
    colp = jnp.pad(col, (0, pad))
    attp = jnp.pad(attention, ((0, pad), (0, 0)))

    coli = colp.reshape(epad // _GRP, _GRP)
    rowi = rowp.reshape(epad // _GRP, _GRP)
    attr = attp.reshape(epad, 2, 4).transpose(1, 0, 2).reshape(2, epad // 4, 16)
    hr = h.reshape(N, 2, 64).transpose(1, 0, 2).reshape(2 * N, 64)

    mesh = plsc.VectorSubcoreMesh(core_axis_name="c", subcore_axis_name="s")
    fn = pl.kernel(
        functools.partial(_body, N, n_blocks),
        out_type=jax.ShapeDtypeStruct((N, HD), jnp.float32),
        mesh=mesh,
        scratch_types=[
            pltpu.VMEM((_GPB, _GRP), jnp.int32),
            pltpu.VMEM((_GPB, _GRP), jnp.int32),
            pltpu.VMEM((_BLK // 4, 16), jnp.float32),
            pltpu.VMEM((_GRP, 64), jnp.float32),
            pltpu.VMEM_SHARED((N, 64), jnp.float32),
            pltpu.SemaphoreType.DMA,
        ],
        compiler_params=pltpu.CompilerParams(use_tc_tiling_on_sc=False),
    )
    return fn(hr, coli, rowi, attr)


# speedup vs baseline: 28.0711x; 1.0881x over previous
import functools

import jax
import jax.numpy as jnp
from jax import lax
from jax.experimental import pallas as pl
from jax.experimental.pallas import tpu as pltpu
from jax.experimental.pallas import tpu_sc as plsc

_NC = 2
_NS = 16
_L = 16

_BLK = 1024
_GRP = 128
_GPB = _BLK // _GRP
_NBUF = 4


def _bcast_lane(v, lane):
    idx = jnp.full((_L, 1), lane, dtype=jnp.int32)
    dn = lax.GatherDimensionNumbers(
        offset_dims=(), collapsed_slice_dims=(0,), start_index_map=(0,))
    return lax.gather(v, idx, dn, (1,),
                      mode=lax.GatherScatterMode.PROMISE_IN_BOUNDS)


def _body(n_nodes, n_blocks, hr, col2, rowi, attr, out,
          idx2, rowb, attb, buf0, buf1, buf2, buf3, acc,
          gs0, gs1, gs2, gs3, ss0, ss1, ss2, ss3):
    c = lax.axis_index("c")
    s = lax.axis_index("s")
    bufs = (buf0, buf1, buf2, buf3)
    gsems = (gs0, gs1, gs2, gs3)
    ssems = (ss0, ss1, ss2, ss3)
    rows_per_tile = n_nodes // _NS
    zchunk = 125

    zero = jnp.zeros((_L,), jnp.float32)

    def zloop(i, _):
        for k in range(4):
            buf0[i, pl.ds(k * _L, _L)] = zero
        return 0
    lax.fori_loop(0, _GRP, zloop, 0)

    r0 = s * rows_per_tile
    for t in range(rows_per_tile // zchunk):
        pltpu.sync_copy(buf0.at[pl.ds(0, zchunk), :],
                        acc.at[pl.ds(r0 + t * zchunk, zchunk), :])
    plsc.subcore_barrier()

    def gather(j, b):
        return pltpu.make_async_copy(hr.at[idx2.at[j]], bufs[b], gsems[b])

    def scatter_start(j, b):
        pltpu.async_copy(bufs[b], acc.at[rowb.at[j]], ssems[b], add=True)

    def scatter_wait(j, b):
        pltpu.make_async_copy(bufs[b], acc.at[rowb.at[j]], ssems[b]).wait()

    def compute(j, b):
        buf = bufs[b]

        def quad_loop(q, _2):
            av = attb[j * (_GRP // 4) + q, :]
            for jj in range(4):
                e = q * 4 + jj
                for hd in range(4):
                    bc = _bcast_lane(av, 4 * jj + hd)
                    sl = pl.ds(hd * _L, _L)
                    buf[e, sl] = buf[e, sl] * bc
            return 0
        lax.fori_loop(0, _GRP // 4, quad_loop, 0)

    def blk_loop(blk, _):
        base128 = s * (n_blocks * _GPB) + blk * _GPB
        qblock = base128 * (_GRP // 4)
        pltpu.sync_copy(col2.at[c, pl.ds(base128, _GPB), :], idx2)
        pltpu.sync_copy(rowi.at[pl.ds(base128, _GPB), :], rowb)
        pltpu.sync_copy(attr.at[c, pl.ds(qblock, _BLK // 4), :], attb)

        gather(0, 0).start()

        def ring_loop(jj, _2):
            for b in range(_NBUF):
                j = jj * _NBUF + b
                nb = (b + 1) % _NBUF

                @pl.when(jnp.logical_and(j >= _NBUF - 1, j < _GPB - 1))
                def _():
                    scatter_wait(j - (_NBUF - 1), nb)

                @pl.when(j < _GPB - 1)
                def _():
                    gather(j + 1, nb).start()

                gather(j, b).wait()
                compute(j, b)
                scatter_start(j, b)
            return 0
        lax.fori_loop(0, _GPB // _NBUF, ring_loop, 0)
        for b in range(_NBUF):
            scatter_wait(_GPB - _NBUF + b, (_GPB - _NBUF + b) % _NBUF)
        return 0
    lax.fori_loop(0, n_blocks, blk_loop, 0)

    plsc.subcore_barrier()
    for t in range(rows_per_tile // zchunk):
        rr = r0 + t * zchunk
        pltpu.sync_copy(acc.at[pl.ds(rr, zchunk), :],
                        out.at[pl.ds(rr, zchunk), pl.ds(c * 64, 64)])


def kernel(attention, h, edge_index):
    E, H = attention.shape
    N, _, D = h.shape
    HD = H * D
    ept = _NS * _BLK
    epad = ((E + ept - 1) // ept) * ept
    n_blocks = epad // ept
    pad = epad - E

    row = edge_index[0].astype(jnp.int32)
    col = edge_index[1].astype(jnp.int32)
    rowp = jnp.pad(row, (0, pad))
    colp = jnp.pad(col, (0, pad))
    attp = jnp.pad(attention, ((0, pad), (0, 0)))

    col2 = jnp.stack([colp, colp + N]).reshape(2, epad // _GRP, _GRP)
    rowi = rowp.reshape(epad // _GRP, _GRP)
    attr = attp.reshape(epad, 2, 4).transpose(1, 0, 2).reshape(2, epad // 4, 16)
    hr = h.reshape(N, 2, 64).transpose(1, 0, 2).reshape(2 * N, 64)

    mesh = plsc.VectorSubcoreMesh(core_axis_name="c", subcore_axis_name="s")
    fn = pl.kernel(
        functools.partial(_body, N, n_blocks),
        out_type=jax.ShapeDtypeStruct((N, HD), jnp.float32),
        mesh=mesh,
        scratch_types=[
            pltpu.VMEM((_GPB, _GRP), jnp.int32),
            pltpu.VMEM((_GPB, _GRP), jnp.int32),
            pltpu.VMEM((_BLK // 4, 16), jnp.float32),
            pltpu.VMEM((_GRP, 64), jnp.float32),
            pltpu.VMEM((_GRP, 64), jnp.float32),
            pltpu.VMEM((_GRP, 64), jnp.float32),
            pltpu.VMEM((_GRP, 64), jnp.float32),
            pltpu.VMEM_SHARED((N, 64), jnp.float32),
            pltpu.SemaphoreType.DMA,
            pltpu.SemaphoreType.DMA,
            pltpu.SemaphoreType.DMA,
            pltpu.SemaphoreType.DMA,
            pltpu.SemaphoreType.DMA,
            pltpu.SemaphoreType.DMA,
            pltpu.SemaphoreType.DMA,
            pltpu.SemaphoreType.DMA,
        ],
        compiler_params=pltpu.CompilerParams(use_tc_tiling_on_sc=False),
    )
    return fn(hr, col2, rowi, attr)
